# baseline (speedup 1.0000x reference)
import functools

import jax
import jax.numpy as jnp
from jax import lax
from jax.experimental import pallas as pl
from jax.experimental.pallas import tpu as pltpu
from jax.experimental.pallas import tpu_sc as plsc

_B, _H, _W, _C = 4096, 22, 22, 3
_NCLS = 11
_NW = 32
_BW = _B // _NW
_WP = 24
_NV = _BW // 16

_mesh = plsc.VectorSubcoreMesh(core_axis_name="c", subcore_axis_name="s")


def _sc_body(xt_hbm, out_hbm, in_buf, oh_buf, sem_out):
    wid = lax.axis_index("s") * 2 + lax.axis_index("c")
    b0 = pl.multiple_of(wid * _BW, _BW)
    iota = lax.iota(jnp.int32, 16)
    ones = jnp.full((16,), 1.0, jnp.float32)
    zeros = jnp.zeros((16,), jnp.float32)

    def _zero(i, carry):
        r = i // _NV
        bb = (i % _NV) * 16
        oh_buf[r, pl.ds(bb, 16)] = zeros
        return carry

    def _put(i, carry):
        w = i // _NV
        bb = (i % _NV) * 16
        xv = in_buf[w, pl.ds(bb, 16)]
        plsc.store_scatter(oh_buf, [xv * _WP + w, iota + bb], ones)
        return carry

    for h in range(_H):
        pltpu.sync_copy(xt_hbm.at[h, 0, :, pl.ds(b0, _BW)], in_buf)
        if h > 0:
            for cp in cps:
                cp.wait()
        lax.fori_loop(0, _NCLS * _WP * _NV, _zero, 0)
        lax.fori_loop(0, _W * _NV, _put, 0)
        cps = [
            pltpu.async_copy(
                oh_buf.at[pl.ds(c * _WP, _W)],
                out_hbm.at[c, h, :, pl.ds(b0, _BW)],
                sem_out,
            )
            for c in range(_NCLS)
        ]
    for cp in cps:
        cp.wait()


@functools.partial(
    pl.kernel,
    mesh=_mesh,
    out_type=jax.ShapeDtypeStruct((_NCLS, _H, _W, _B), jnp.float32),
    scratch_types=[
        pltpu.VMEM((_W, _BW), jnp.int32),
        pltpu.VMEM((_NCLS * _WP, _BW), jnp.float32),
        pltpu.SemaphoreType.DMA,
    ],
    compiler_params=pltpu.CompilerParams(
        use_tc_tiling_on_sc=True, needs_layout_passes=False
    ),
)
def _sc_kernel(xt_hbm, out_hbm, in_buf, oh_buf, sem_out):
    _sc_body(xt_hbm, out_hbm, in_buf, oh_buf, sem_out)


def kernel(x):
    xt = jnp.transpose(x, (1, 3, 2, 0))
    ot = _sc_kernel(xt)
    return jnp.transpose(ot, (3, 0, 1, 2))


# speedup vs baseline: 2.2572x; 2.2572x over previous
import functools

import jax
import jax.numpy as jnp
from jax import lax
from jax.experimental import pallas as pl
from jax.experimental.pallas import tpu as pltpu
from jax.experimental.pallas import tpu_sc as plsc

_B, _H, _W, _C = 4096, 22, 22, 3
_NCLS = 11
_NW = 32
_BW = _B // _NW
_WP = 24
_NV = _BW // 16

_mesh = plsc.VectorSubcoreMesh(core_axis_name="c", subcore_axis_name="s")


def _sc_body(xt_hbm, out_hbm, in_a, in_b, oh_buf, sem_in, sem_out):
    wid = lax.axis_index("s") * 2 + lax.axis_index("c")
    b0 = pl.multiple_of(wid * _BW, _BW)
    iota = lax.iota(jnp.int32, 16)
    ones = jnp.full((16,), 1.0, jnp.float32)
    zeros = jnp.zeros((16,), jnp.float32)

    def _zero(r, carry):
        for bb in range(_NV):
            oh_buf[r, pl.ds(bb * 16, 16)] = zeros
        return carry

    lax.fori_loop(0, _NCLS * _WP, _zero, 0)

    def _scatter(src_ref, val):
        def body(w, carry):
            for bb in range(_NV):
                xv = src_ref[w, pl.ds(bb * 16, 16)]
                plsc.store_scatter(oh_buf, [xv * _WP + w, iota + bb * 16], val)
            return carry

        lax.fori_loop(0, _W, body, 0)

    bufs = [in_a, in_b]
    cps = None
    cp_in = pltpu.async_copy(xt_hbm.at[0, 0, :, pl.ds(b0, _BW)], in_a, sem_in)
    for h in range(_H):
        cur = bufs[h % 2]
        prev = bufs[(h + 1) % 2]
        cp_in.wait()
        if cps is not None:
            for cp in cps:
                cp.wait()
        if h > 0:
            _scatter(prev, zeros)
        if h + 1 < _H:
            cp_in = pltpu.async_copy(
                xt_hbm.at[h + 1, 0, :, pl.ds(b0, _BW)], prev, sem_in
            )
        _scatter(cur, ones)
        cps = [
            pltpu.async_copy(
                oh_buf.at[pl.ds(c * _WP, _W)],
                out_hbm.at[c, h, :, pl.ds(b0, _BW)],
                sem_out,
            )
            for c in range(_NCLS)
        ]
    for cp in cps:
        cp.wait()


@functools.partial(
    pl.kernel,
    mesh=_mesh,
    out_type=jax.ShapeDtypeStruct((_NCLS, _H, _W, _B), jnp.float32),
    scratch_types=[
        pltpu.VMEM((_W, _BW), jnp.int32),
        pltpu.VMEM((_W, _BW), jnp.int32),
        pltpu.VMEM((_NCLS * _WP, _BW), jnp.float32),
        pltpu.SemaphoreType.DMA,
        pltpu.SemaphoreType.DMA,
    ],
    compiler_params=pltpu.CompilerParams(
        use_tc_tiling_on_sc=True, needs_layout_passes=False
    ),
)
def _sc_kernel(xt_hbm, out_hbm, in_a, in_b, oh_buf, sem_in, sem_out):
    _sc_body(xt_hbm, out_hbm, in_a, in_b, oh_buf, sem_in, sem_out)


def kernel(x):
    xt = jnp.transpose(x, (1, 3, 2, 0))
    ot = _sc_kernel(xt)
    return jnp.transpose(ot, (3, 0, 1, 2))


# speedup vs baseline: 2.8114x; 1.2456x over previous
import functools

import jax
import jax.numpy as jnp
from jax import lax
from jax.experimental import pallas as pl
from jax.experimental.pallas import tpu as pltpu
from jax.experimental.pallas import tpu_sc as plsc

_B, _H, _W, _C = 4096, 22, 22, 3
_NCLS = 11
_NW = 32
_BW = _B // _NW
_WP = 24
_NV = _BW // 16

_mesh = plsc.VectorSubcoreMesh(core_axis_name="c", subcore_axis_name="s")


def _sc_body(xt_hbm, out_hbm, in_a, in_b, in_c, oh_a, oh_b, sem_in, sem_out):
    wid = lax.axis_index("s") * 2 + lax.axis_index("c")
    b0 = pl.multiple_of(wid * _BW, _BW)
    iota = lax.iota(jnp.int32, 16)
    ones = jnp.full((16,), 1.0, jnp.float32)
    zeros = jnp.zeros((16,), jnp.float32)
    in_bufs = [in_a, in_b, in_c]
    oh_bufs = [oh_a, oh_b]

    def _zero(oh_buf):
        def body(r, carry):
            for bb in range(_NV):
                oh_buf[r, pl.ds(bb * 16, 16)] = zeros
            return carry

        lax.fori_loop(0, _NCLS * _WP, body, 0)

    _zero(oh_a)
    _zero(oh_b)

    def _scatter(oh_buf, src_ref, val):
        def body(w, carry):
            for bb in range(_NV):
                xv = src_ref[w, pl.ds(bb * 16, 16)]
                plsc.store_scatter(oh_buf, [xv * _WP + w, iota + bb * 16], val)
            return carry

        lax.fori_loop(0, _W, body, 0)

    cps = {}
    cp_in = pltpu.async_copy(xt_hbm.at[0, 0, :, pl.ds(b0, _BW)], in_a, sem_in)
    for h in range(_H):
        cur = in_bufs[h % 3]
        oh = oh_bufs[h % 2]
        cp_in.wait()
        if h + 1 < _H:
            cp_in = pltpu.async_copy(
                xt_hbm.at[h + 1, 0, :, pl.ds(b0, _BW)], in_bufs[(h + 1) % 3], sem_in
            )
        if h >= 2:
            for cp in cps[h - 2]:
                cp.wait()
            _scatter(oh, in_bufs[(h - 2) % 3], zeros)
        _scatter(oh, cur, ones)
        cps[h] = [
            pltpu.async_copy(
                oh.at[pl.ds(c * _WP, _W)],
                out_hbm.at[c, h, :, pl.ds(b0, _BW)],
                sem_out,
            )
            for c in range(_NCLS)
        ]
    for h in (_H - 2, _H - 1):
        for cp in cps[h]:
            cp.wait()


@functools.partial(
    pl.kernel,
    mesh=_mesh,
    out_type=jax.ShapeDtypeStruct((_NCLS, _H, _W, _B), jnp.float32),
    scratch_types=[
        pltpu.VMEM((_W, _BW), jnp.int32),
        pltpu.VMEM((_W, _BW), jnp.int32),
        pltpu.VMEM((_W, _BW), jnp.int32),
        pltpu.VMEM((_NCLS * _WP, _BW), jnp.float32),
        pltpu.VMEM((_NCLS * _WP, _BW), jnp.float32),
        pltpu.SemaphoreType.DMA,
        pltpu.SemaphoreType.DMA,
    ],
    compiler_params=pltpu.CompilerParams(
        use_tc_tiling_on_sc=True, needs_layout_passes=False
    ),
)
def _sc_kernel(xt_hbm, out_hbm, in_a, in_b, in_c, oh_a, oh_b, sem_in, sem_out):
    _sc_body(xt_hbm, out_hbm, in_a, in_b, in_c, oh_a, oh_b, sem_in, sem_out)


def kernel(x):
    xt = jnp.transpose(x, (1, 3, 2, 0))
    ot = _sc_kernel(xt)
    return jnp.transpose(ot, (3, 0, 1, 2))
